# baseline (speedup 1.0000x reference)
import jax
import jax.numpy as jnp
from jax import lax
from jax.experimental import pallas as pl
from jax.experimental.pallas import tpu as pltpu
from jax.experimental.pallas import tpu_sc as plsc

P = 16
D = 1024
N_CTX = 8192
VOCAB = 258
PAD_ID = 257
NB = 4

_info = plsc.get_sparse_core_info()
NC, NS, L = _info.num_cores, _info.num_subcores, _info.num_lanes
NW = NC * NS

JSLAB = N_CTX // NW
CHUNK = P
STEPS = (JSLAB // CHUNK) * NB
NRB = 3


def _sc_body(ids_hbm, byte_hbm, pos_hbm, out_hbm,
             idx_v, rows_v, pos_v, outb_v, gsem, psem, osem):
    sid = lax.axis_index("s")
    wid = sid * NC + lax.axis_index("c")
    jbase = wid * JSLAB
    kbase = jbase // P

    def gather(t):
        b = lax.rem(t, NB)
        c = lax.div(t, NB)
        idx = idx_v.at[pl.ds(b * JSLAB + c * CHUNK, CHUNK)]
        pltpu.async_copy(byte_hbm.at[idx], rows_v.at[lax.rem(t, NRB)], gsem)

    for b in range(NB):
        pltpu.sync_copy(ids_hbm.at[pl.ds(b * N_CTX + jbase, JSLAB)],
                        idx_v.at[pl.ds(b * JSLAB, JSLAB)])

    pltpu.async_copy(pos_hbm.at[pl.ds(jbase, CHUNK)], pos_v.at[0], psem)
    gather(0)

    def step(t, _):
        b = lax.rem(t, NB)
        c = lax.div(t, NB)
        rbuf = lax.rem(t, NRB)
        obuf = lax.rem(t, 2)
        pbuf = lax.rem(c, 2)

        @pl.when(t < STEPS - 1)
        def _():
            gather(t + 1)

        @pl.when(b == 0)
        def _():
            pltpu.make_async_copy(pos_hbm.at[pl.ds(0, CHUNK)], pos_v.at[0],
                                  psem).wait()

        @pl.when(jnp.logical_and(b == NB - 1, t < STEPS - 1))
        def _():
            pltpu.async_copy(pos_hbm.at[pl.ds(jbase + (c + 1) * CHUNK, CHUNK)],
                             pos_v.at[lax.rem(c + 1, 2)], psem)

        @pl.when(t >= 2)
        def _():
            pltpu.make_async_copy(outb_v.at[0],
                                  out_hbm.at[0, pl.ds(0, 1)], osem).wait()

        pltpu.make_async_copy(byte_hbm.at[idx_v.at[pl.ds(0, CHUNK)]],
                              rows_v.at[0], gsem).wait()
        rv = rows_v.at[rbuf]
        pv = pos_v.at[pbuf]
        ov = outb_v.at[obuf]

        @plsc.parallel_loop(0, CHUNK, unroll=4)
        def add_row(i):
            for k in range(D // L):
                col = k * L
                ov[0, pl.ds(i * D + col, L)] = (
                    rv[i, pl.ds(col, L)] + pv[i, pl.ds(col, L)]
                )

        pltpu.async_copy(ov, out_hbm.at[b, pl.ds(kbase + c, 1)], osem)
        return 0

    lax.fori_loop(0, STEPS, step, 0)
    for _ in range(2):
        pltpu.make_async_copy(outb_v.at[0], out_hbm.at[0, pl.ds(0, 1)],
                              osem).wait()


@jax.jit
def kernel(x_bytes, byte_embed, global_pos_embed):
    B, T = x_bytes.shape
    K = T // P
    pad = jnp.full((B, P), PAD_ID, dtype=jnp.int32)
    ids = jnp.concatenate([pad, x_bytes[:, : T - P].astype(jnp.int32)], axis=1)
    ids_flat = ids.reshape(B * T)

    mesh = plsc.VectorSubcoreMesh(core_axis_name="c", subcore_axis_name="s")
    run = pl.kernel(
        _sc_body,
        mesh=mesh,
        out_type=jax.ShapeDtypeStruct((B, K, P * D), jnp.float32),
        scratch_types=[
            pltpu.VMEM((NB * JSLAB,), jnp.int32),
            pltpu.VMEM((NRB, CHUNK, D), jnp.float32),
            pltpu.VMEM((2, CHUNK, D), jnp.float32),
            pltpu.VMEM((2, 1, P * D), jnp.float32),
            pltpu.SemaphoreType.DMA,
            pltpu.SemaphoreType.DMA,
            pltpu.SemaphoreType.DMA,
        ],
    )
    return run(ids_flat, byte_embed, global_pos_embed)


# speedup vs baseline: 1.2521x; 1.2521x over previous
import jax
import jax.numpy as jnp
from jax import lax
from jax.experimental import pallas as pl
from jax.experimental.pallas import tpu as pltpu
from jax.experimental.pallas import tpu_sc as plsc

P = 16
D = 1024
N_CTX = 8192
VOCAB = 258
PAD_ID = 257
NB = 4

_info = plsc.get_sparse_core_info()
NC, NS, L = _info.num_cores, _info.num_subcores, _info.num_lanes
NW = NC * NS

JSLAB = N_CTX // NW
CHUNK = P
STEPS = (JSLAB // CHUNK) * NB
NRB = 3


def _sc_body(ids_hbm, byte_hbm, pos_hbm, out_hbm,
             idx_v, rows_v, pos_v, outb_v, gsem, psem, osem):
    sid = lax.axis_index("s")
    wid = sid * NC + lax.axis_index("c")
    jbase = wid * JSLAB
    kbase = jbase // P

    def gather(t):
        b = lax.rem(t, NB)
        c = lax.div(t, NB)
        idx = idx_v.at[pl.ds(b * JSLAB + c * CHUNK, CHUNK)]
        pltpu.async_copy(byte_hbm.at[idx], rows_v.at[lax.rem(t, NRB)], gsem)

    for b in range(NB):
        pltpu.sync_copy(ids_hbm.at[pl.ds(b * N_CTX + jbase, JSLAB)],
                        idx_v.at[pl.ds(b * JSLAB, JSLAB)])

    pltpu.async_copy(pos_hbm.at[pl.ds(jbase, CHUNK)], pos_v.at[0], psem)
    gather(0)

    def step(t, _):
        b = lax.rem(t, NB)
        c = lax.div(t, NB)
        rbuf = lax.rem(t, NRB)
        obuf = lax.rem(t, 2)
        pbuf = lax.rem(c, 2)

        @pl.when(t < STEPS - 1)
        def _():
            gather(t + 1)

        @pl.when(b == 0)
        def _():
            pltpu.make_async_copy(pos_hbm.at[pl.ds(0, CHUNK)], pos_v.at[0],
                                  psem).wait()

        @pl.when(jnp.logical_and(b == NB - 1, t < STEPS - 1))
        def _():
            pltpu.async_copy(pos_hbm.at[pl.ds(jbase + (c + 1) * CHUNK, CHUNK)],
                             pos_v.at[lax.rem(c + 1, 2)], psem)

        @pl.when(t >= 2)
        def _():
            pltpu.make_async_copy(outb_v.at[0],
                                  out_hbm.at[0, pl.ds(0, 1)], osem).wait()

        pltpu.make_async_copy(byte_hbm.at[idx_v.at[pl.ds(0, CHUNK)]],
                              rows_v.at[0], gsem).wait()
        rv = rows_v.at[rbuf]
        pv = pos_v.at[pbuf]
        ov = outb_v.at[obuf]

        @plsc.parallel_loop(0, CHUNK, unroll=2)
        def add_row(i):
            for k in range(D // L):
                col = k * L
                ov[0, pl.ds(i * D + col, L)] = (
                    rv[i, pl.ds(col, L)] + pv[i, pl.ds(col, L)]
                )

        pltpu.async_copy(ov, out_hbm.at[b, pl.ds(kbase + c, 1)], osem)
        return 0

    lax.fori_loop(0, STEPS, step, 0)
    for _ in range(2):
        pltpu.make_async_copy(outb_v.at[0], out_hbm.at[0, pl.ds(0, 1)],
                              osem).wait()


@jax.jit
def kernel(x_bytes, byte_embed, global_pos_embed):
    B, T = x_bytes.shape
    K = T // P
    pad = jnp.full((B, P), PAD_ID, dtype=jnp.int32)
    ids = jnp.concatenate([pad, x_bytes[:, : T - P].astype(jnp.int32)], axis=1)
    ids_flat = ids.reshape(B * T)

    mesh = plsc.VectorSubcoreMesh(core_axis_name="c", subcore_axis_name="s")
    run = pl.kernel(
        _sc_body,
        mesh=mesh,
        out_type=jax.ShapeDtypeStruct((B, K, P * D), jnp.float32),
        scratch_types=[
            pltpu.VMEM((NB * JSLAB,), jnp.int32),
            pltpu.VMEM((NRB, CHUNK, D), jnp.float32),
            pltpu.VMEM((2, CHUNK, D), jnp.float32),
            pltpu.VMEM((2, 1, P * D), jnp.float32),
            pltpu.SemaphoreType.DMA,
            pltpu.SemaphoreType.DMA,
            pltpu.SemaphoreType.DMA,
        ],
    )
    return run(ids_flat, byte_embed, global_pos_embed)
